# baseline (speedup 1.0000x reference)
import functools

import jax
import jax.numpy as jnp
from jax import lax
from jax.experimental import pallas as pl
from jax.experimental.pallas import tpu as pltpu
from jax.experimental.pallas import tpu_sc as plsc

_B, _H, _D, _V = 16384, 50, 64, 1000000
_NC, _NS, _L = 2, 16, 16
_NW = _NC * _NS
_BPW = _B // _NW
_C = 4
_CW = _C * _H
_NCH = _BPW // _C
_INV = 1.0 / _H
_WIDX = _BPW * _H


def _body(xs_hbm, tab_hbm, out_hbm, xs_v, idxh0, idxh1, rows0, rows1,
          out_v, sem0, sem1):
    wid = lax.axis_index("s") * _NC + lax.axis_index("c")
    base = wid * _BPW

    pltpu.sync_copy(xs_hbm.at[pl.ds(base * _H, _WIDX)], xs_v.at[pl.ds(0, _WIDX)])

    def compute_half_ids(cbase, idxh):
        for k in (0, 16, 32, 48, 64, 80, 96, 112, 128, 144, 160, 176, 184):
            v = xs_v[pl.ds(cbase + k, _L)]
            idxh[pl.ds(k, _L)] = jax.lax.shift_right_logical(v, 1)

    compute_half_ids(0, idxh0)
    pltpu.async_copy(tab_hbm.at[idxh0], rows0, sem0)
    compute_half_ids(_CW, idxh1)
    pltpu.async_copy(tab_hbm.at[idxh1], rows1, sem1)

    @pl.loop(0, _NCH, step=2)
    def _chunks(ci):
        for b in range(2):
            rows = rows0 if b == 0 else rows1
            idxh = idxh0 if b == 0 else idxh1
            sem = sem0 if b == 0 else sem1
            cur = ci + b
            pltpu.make_async_copy(tab_hbm.at[idxh], rows, sem).wait()
            cbase = cur * _CW

            @pl.loop(0, _C)
            def _items(i, rows=rows, cur=cur, cbase=cbase):
                rowbase = i * _H
                z = jnp.zeros((_L,), jnp.float32)

                @pl.loop(0, _H, init_carry=(z, z, z, z), unroll=2)
                def _acc(j, carry, rows=rows, rowbase=rowbase, cbase=cbase):
                    a0, a1, a2, a3 = carry
                    rr = rowbase + j
                    pid = xs_v[pl.ds(cbase + rr, _L)][0]
                    p = (pid & 1) * _D
                    return (
                        a0 + rows[rr, pl.ds(p, _L)],
                        a1 + rows[rr, pl.ds(p + _L, _L)],
                        a2 + rows[rr, pl.ds(p + 2 * _L, _L)],
                        a3 + rows[rr, pl.ds(p + 3 * _L, _L)],
                    )

                a0, a1, a2, a3 = _acc
                r0 = cur * _C + i
                q = r0 >> 1
                p = (r0 & 1) * _D
                out_v[q, pl.ds(p, _L)] = a0 * _INV
                out_v[q, pl.ds(p + _L, _L)] = a1 * _INV
                out_v[q, pl.ds(p + 2 * _L, _L)] = a2 * _INV
                out_v[q, pl.ds(p + 3 * _L, _L)] = a3 * _INV

            nxt = cur + 2

            @pl.when(nxt < _NCH)
            def _fire(rows=rows, idxh=idxh, sem=sem, nxt=nxt):
                compute_half_ids(nxt * _CW, idxh)
                pltpu.async_copy(tab_hbm.at[idxh], rows, sem)

    pltpu.sync_copy(out_v, out_hbm.at[pl.ds(wid * (_BPW // 2), _BPW // 2)])


@functools.cache
def _make_kernel():
    mesh = plsc.VectorSubcoreMesh(
        core_axis_name="c", subcore_axis_name="s",
        num_cores=_NC, num_subcores=_NS,
    )
    return pl.kernel(
        _body,
        out_type=jax.ShapeDtypeStruct((_B // 2, 2 * _D), jnp.float32),
        mesh=mesh,
        scratch_types=[
            pltpu.VMEM((_WIDX + _L,), jnp.int32),
            pltpu.VMEM((_CW,), jnp.int32),
            pltpu.VMEM((_CW,), jnp.int32),
            pltpu.VMEM((_CW, 2 * _D), jnp.float32),
            pltpu.VMEM((_CW, 2 * _D), jnp.float32),
            pltpu.VMEM((_BPW // 2, 2 * _D), jnp.float32),
            pltpu.SemaphoreType.DMA,
            pltpu.SemaphoreType.DMA,
        ],
        compiler_params=pltpu.CompilerParams(use_tc_tiling_on_sc=True),
    )


def kernel(xs, table):
    xs_flat = xs.reshape(-1).astype(jnp.int32)
    tab_packed = table.reshape(_V // 2, 2 * _D)
    out_packed = _make_kernel()(xs_flat, tab_packed)
    return out_packed.reshape(_B, _D)


# speedup vs baseline: 1.7456x; 1.7456x over previous
import functools

import jax
import jax.numpy as jnp
from jax import lax
from jax.experimental import pallas as pl
from jax.experimental.pallas import tpu as pltpu
from jax.experimental.pallas import tpu_sc as plsc

_B, _H, _D, _V = 16384, 50, 64, 1000000
_BR = 2048
_NB = 245
_S = _BR * _NB
_VP = 2 * _S

_NC, _NS, _L = 2, 16, 16
_NW = _NC * _NS
_BPW = _B // _NW
_C = 8
_CW = _C * _H
_NCH = _BPW // _C
_INV = 1.0 / _H
_WIDX = _BPW * _H


def _pack_body(x1_ref, x2_ref, out_ref):
    y1 = jnp.swapaxes(x1_ref[...], 0, 1)
    y2 = jnp.swapaxes(x2_ref[...], 0, 1)
    out_ref[...] = jnp.concatenate([y1, y2], axis=1)


def _pack(tab_t):
    return pl.pallas_call(
        _pack_body,
        out_shape=jax.ShapeDtypeStruct((_S, 2 * _D), jnp.float32),
        grid=(_NB,),
        in_specs=[
            pl.BlockSpec((_D, _BR), lambda g: (0, g)),
            pl.BlockSpec(
                (_D, _BR),
                lambda g: (0, jnp.minimum(g + _NB, (_V + _BR - 1) // _BR - 1)),
            ),
        ],
        out_specs=pl.BlockSpec((_BR, 2 * _D), lambda g: (g, 0)),
    )(tab_t, tab_t)


def _body(xs_hbm, tab_hbm, out_hbm, xs_v, rows0, rows1, out_v, sem0, sem1):
    wid = lax.axis_index("s") * _NC + lax.axis_index("c")
    base = wid * _BPW

    pltpu.sync_copy(xs_hbm.at[pl.ds(base * _H, _WIDX)], xs_v)

    @pl.loop(0, _WIDX, step=_L)
    def _remap(k):
        v = xs_v[pl.ds(k, _L)]
        ge = v >= _S
        xs_v[pl.ds(k, _L)] = 2 * v - jnp.where(ge, _VP - 1, 0)

    pltpu.async_copy(tab_hbm.at[xs_v.at[pl.ds(0, _CW)]], rows0, sem0)
    pltpu.async_copy(tab_hbm.at[xs_v.at[pl.ds(_CW, _CW)]], rows1, sem1)

    @pl.loop(0, _NCH, step=2)
    def _chunks(ci):
        for b in range(2):
            rows = rows0 if b == 0 else rows1
            sem = sem0 if b == 0 else sem1
            cur = ci + b
            pltpu.make_async_copy(
                tab_hbm.at[xs_v.at[pl.ds(cur * _CW, _CW)]], rows, sem
            ).wait()

            @pl.loop(0, _C)
            def _items(i, rows=rows, cur=cur):
                rowbase = i * _H
                z = jnp.zeros((_L,), jnp.float32)

                @pl.loop(0, _H, init_carry=(z, z, z, z), unroll=2)
                def _acc(j, carry, rows=rows, rowbase=rowbase):
                    a0, a1, a2, a3 = carry
                    rr = rowbase + j
                    return (
                        a0 + rows[rr, pl.ds(0, _L)],
                        a1 + rows[rr, pl.ds(_L, _L)],
                        a2 + rows[rr, pl.ds(2 * _L, _L)],
                        a3 + rows[rr, pl.ds(3 * _L, _L)],
                    )

                a0, a1, a2, a3 = _acc
                r0 = cur * _C + i
                out_v[r0, pl.ds(0, _L)] = a0 * _INV
                out_v[r0, pl.ds(_L, _L)] = a1 * _INV
                out_v[r0, pl.ds(2 * _L, _L)] = a2 * _INV
                out_v[r0, pl.ds(3 * _L, _L)] = a3 * _INV

            nxt = cur + 2

            @pl.when(nxt < _NCH)
            def _fire(rows=rows, sem=sem, nxt=nxt):
                pltpu.async_copy(
                    tab_hbm.at[xs_v.at[pl.ds(nxt * _CW, _CW)]], rows, sem
                )

    pltpu.sync_copy(out_v, out_hbm.at[pl.ds(base, _BPW)])


@functools.cache
def _make_sc_kernel():
    mesh = plsc.VectorSubcoreMesh(
        core_axis_name="c", subcore_axis_name="s",
        num_cores=_NC, num_subcores=_NS,
    )
    return pl.kernel(
        _body,
        out_type=jax.ShapeDtypeStruct((_B, _D), jnp.float32),
        mesh=mesh,
        scratch_types=[
            pltpu.VMEM((_WIDX,), jnp.int32),
            pltpu.VMEM((_CW, _D), jnp.float32),
            pltpu.VMEM((_CW, _D), jnp.float32),
            pltpu.VMEM((_BPW, _D), jnp.float32),
            pltpu.SemaphoreType.DMA,
            pltpu.SemaphoreType.DMA,
        ],
        compiler_params=pltpu.CompilerParams(use_tc_tiling_on_sc=False),
    )


def kernel(xs, table):
    xs_flat = xs.reshape(-1).astype(jnp.int32)
    tab_lin = _pack(table.T).reshape(_VP, _D)
    return _make_sc_kernel()(xs_flat, tab_lin)


# speedup vs baseline: 1.9839x; 1.1365x over previous
import functools

import jax
import jax.numpy as jnp
from jax import lax
from jax.experimental import pallas as pl
from jax.experimental.pallas import tpu as pltpu
from jax.experimental.pallas import tpu_sc as plsc

_B, _H, _D, _V = 16384, 50, 64, 1000000
_BR = 2048
_NB = 245
_S = _BR * _NB
_VP = 2 * _S

_NC, _NS, _L = 2, 16, 16
_NW = _NC * _NS
_BPW = _B // _NW
_C = 8
_CW = _C * _H
_NCH = _BPW // _C
_INV = 1.0 / _H
_WIDX = _BPW * _H


def _pack_body(x1_ref, x2_ref, out_ref):
    x = jnp.concatenate([x1_ref[...], x2_ref[...]], axis=0)
    out_ref[...] = jnp.swapaxes(x, 0, 1)


def _pack(tab_t):
    return pl.pallas_call(
        _pack_body,
        out_shape=jax.ShapeDtypeStruct((_S, 2 * _D), jnp.float32),
        grid=(_NB,),
        in_specs=[
            pl.BlockSpec((_D, _BR), lambda g: (0, g)),
            pl.BlockSpec(
                (_D, _BR),
                lambda g: (0, jnp.minimum(g + _NB, (_V + _BR - 1) // _BR - 1)),
            ),
        ],
        out_specs=pl.BlockSpec((_BR, 2 * _D), lambda g: (g, 0)),
    )(tab_t, tab_t)


def _body(xs_hbm, tab_hbm, out_hbm, xs_v, rows0, rows1, out_v, sem0, sem1):
    wid = lax.axis_index("s") * _NC + lax.axis_index("c")
    base = wid * _BPW

    pltpu.sync_copy(xs_hbm.at[pl.ds(base * _H, _WIDX)], xs_v)

    @pl.loop(0, _WIDX, step=_L)
    def _remap(k):
        v = xs_v[pl.ds(k, _L)]
        ge = v >= _S
        xs_v[pl.ds(k, _L)] = 2 * v - jnp.where(ge, _VP - 1, 0)

    pltpu.async_copy(tab_hbm.at[xs_v.at[pl.ds(0, _CW)]], rows0, sem0)
    pltpu.async_copy(tab_hbm.at[xs_v.at[pl.ds(_CW, _CW)]], rows1, sem1)

    @pl.loop(0, _NCH, step=2)
    def _chunks(ci):
        for b in range(2):
            rows = rows0 if b == 0 else rows1
            sem = sem0 if b == 0 else sem1
            cur = ci + b
            pltpu.make_async_copy(
                tab_hbm.at[xs_v.at[pl.ds(cur * _CW, _CW)]], rows, sem
            ).wait()

            @pl.loop(0, _C)
            def _items(i, rows=rows, cur=cur):
                rowbase = i * _H
                z = jnp.zeros((_L,), jnp.float32)

                @pl.loop(0, _H, init_carry=(z, z, z, z), unroll=2)
                def _acc(j, carry, rows=rows, rowbase=rowbase):
                    a0, a1, a2, a3 = carry
                    rr = rowbase + j
                    return (
                        a0 + rows[rr, pl.ds(0, _L)],
                        a1 + rows[rr, pl.ds(_L, _L)],
                        a2 + rows[rr, pl.ds(2 * _L, _L)],
                        a3 + rows[rr, pl.ds(3 * _L, _L)],
                    )

                a0, a1, a2, a3 = _acc
                r0 = cur * _C + i
                out_v[r0, pl.ds(0, _L)] = a0 * _INV
                out_v[r0, pl.ds(_L, _L)] = a1 * _INV
                out_v[r0, pl.ds(2 * _L, _L)] = a2 * _INV
                out_v[r0, pl.ds(3 * _L, _L)] = a3 * _INV

            nxt = cur + 2

            @pl.when(nxt < _NCH)
            def _fire(rows=rows, sem=sem, nxt=nxt):
                pltpu.async_copy(
                    tab_hbm.at[xs_v.at[pl.ds(nxt * _CW, _CW)]], rows, sem
                )

    pltpu.sync_copy(out_v, out_hbm.at[pl.ds(base, _BPW)])


@functools.cache
def _make_sc_kernel():
    mesh = plsc.VectorSubcoreMesh(
        core_axis_name="c", subcore_axis_name="s",
        num_cores=_NC, num_subcores=_NS,
    )
    return pl.kernel(
        _body,
        out_type=jax.ShapeDtypeStruct((_B, _D), jnp.float32),
        mesh=mesh,
        scratch_types=[
            pltpu.VMEM((_WIDX,), jnp.int32),
            pltpu.VMEM((_CW, _D), jnp.float32),
            pltpu.VMEM((_CW, _D), jnp.float32),
            pltpu.VMEM((_BPW, _D), jnp.float32),
            pltpu.SemaphoreType.DMA,
            pltpu.SemaphoreType.DMA,
        ],
        compiler_params=pltpu.CompilerParams(use_tc_tiling_on_sc=False),
    )


def kernel(xs, table):
    xs_flat = xs.reshape(-1).astype(jnp.int32)
    tab_lin = _pack(table.T).reshape(_VP, _D)
    return _make_sc_kernel()(xs_flat, tab_lin)


# speedup vs baseline: 2.4134x; 1.2165x over previous
import functools

import jax
import jax.numpy as jnp
from jax import lax
from jax.experimental import pallas as pl
from jax.experimental.pallas import tpu as pltpu
from jax.experimental.pallas import tpu_sc as plsc

_B, _H, _D, _V = 16384, 50, 64, 1000000
_BR = 4096
_NB = 123
_S = _BR * _NB
_VP = 2 * _S

_NC, _NS, _L = 2, 16, 16
_NW = _NC * _NS
_BPW = _B // _NW
_C = 8
_CW = _C * _H
_NCH = _BPW // _C
_INV = 1.0 / _H
_WIDX = _BPW * _H


def _pack_body(x1_ref, x2_ref, out_ref):
    x = jnp.concatenate([x1_ref[...], x2_ref[...]], axis=0)
    out_ref[...] = jnp.swapaxes(x, 0, 1)


def _pack(tab_t):
    return pl.pallas_call(
        _pack_body,
        out_shape=jax.ShapeDtypeStruct((_S, 2 * _D), jnp.float32),
        grid=(_NB,),
        in_specs=[
            pl.BlockSpec((_D, _BR), lambda g: (0, g)),
            pl.BlockSpec(
                (_D, _BR),
                lambda g: (0, jnp.minimum(g + _NB, (_V + _BR - 1) // _BR - 1)),
            ),
        ],
        out_specs=pl.BlockSpec((_BR, 2 * _D), lambda g: (g, 0)),
    )(tab_t, tab_t)


def _body(xs_hbm, tab_hbm, out_hbm, xs_v, rows0, rows1, out_v, sem0, sem1):
    wid = lax.axis_index("s") * _NC + lax.axis_index("c")
    base = wid * _BPW

    pltpu.sync_copy(xs_hbm.at[pl.ds(base * _H, _WIDX)], xs_v)

    @pl.loop(0, _WIDX, step=_L)
    def _remap(k):
        v = xs_v[pl.ds(k, _L)]
        ge = v >= _S
        xs_v[pl.ds(k, _L)] = 2 * v - jnp.where(ge, _VP - 1, 0)

    pltpu.async_copy(tab_hbm.at[xs_v.at[pl.ds(0, _CW)]], rows0, sem0)
    pltpu.async_copy(tab_hbm.at[xs_v.at[pl.ds(_CW, _CW)]], rows1, sem1)

    @pl.loop(0, _NCH, step=2)
    def _chunks(ci):
        for b in range(2):
            rows = rows0 if b == 0 else rows1
            sem = sem0 if b == 0 else sem1
            cur = ci + b
            pltpu.make_async_copy(
                tab_hbm.at[xs_v.at[pl.ds(cur * _CW, _CW)]], rows, sem
            ).wait()

            @pl.loop(0, _C)
            def _items(i, rows=rows, cur=cur):
                rowbase = i * _H
                z = jnp.zeros((_L,), jnp.float32)

                @pl.loop(0, _H, init_carry=(z, z, z, z), unroll=2)
                def _acc(j, carry, rows=rows, rowbase=rowbase):
                    a0, a1, a2, a3 = carry
                    rr = rowbase + j
                    return (
                        a0 + rows[rr, pl.ds(0, _L)],
                        a1 + rows[rr, pl.ds(_L, _L)],
                        a2 + rows[rr, pl.ds(2 * _L, _L)],
                        a3 + rows[rr, pl.ds(3 * _L, _L)],
                    )

                a0, a1, a2, a3 = _acc
                r0 = cur * _C + i
                out_v[r0, pl.ds(0, _L)] = a0 * _INV
                out_v[r0, pl.ds(_L, _L)] = a1 * _INV
                out_v[r0, pl.ds(2 * _L, _L)] = a2 * _INV
                out_v[r0, pl.ds(3 * _L, _L)] = a3 * _INV

            nxt = cur + 2

            @pl.when(nxt < _NCH)
            def _fire(rows=rows, sem=sem, nxt=nxt):
                pltpu.async_copy(
                    tab_hbm.at[xs_v.at[pl.ds(nxt * _CW, _CW)]], rows, sem
                )

    pltpu.sync_copy(out_v, out_hbm.at[pl.ds(base, _BPW)])


@functools.cache
def _make_sc_kernel():
    mesh = plsc.VectorSubcoreMesh(
        core_axis_name="c", subcore_axis_name="s",
        num_cores=_NC, num_subcores=_NS,
    )
    return pl.kernel(
        _body,
        out_type=jax.ShapeDtypeStruct((_B, _D), jnp.float32),
        mesh=mesh,
        scratch_types=[
            pltpu.VMEM((_WIDX,), jnp.int32),
            pltpu.VMEM((_CW, _D), jnp.float32),
            pltpu.VMEM((_CW, _D), jnp.float32),
            pltpu.VMEM((_BPW, _D), jnp.float32),
            pltpu.SemaphoreType.DMA,
            pltpu.SemaphoreType.DMA,
        ],
        compiler_params=pltpu.CompilerParams(use_tc_tiling_on_sc=False),
    )


def kernel(xs, table):
    xs_flat = xs.reshape(-1).astype(jnp.int32)
    tab_lin = _pack(table.T).reshape(_VP, _D)
    return _make_sc_kernel()(xs_flat, tab_lin)


# speedup vs baseline: 2.6168x; 1.0843x over previous
import functools

import jax
import jax.numpy as jnp
from jax import lax
from jax.experimental import pallas as pl
from jax.experimental.pallas import tpu as pltpu
from jax.experimental.pallas import tpu_sc as plsc

_B, _H, _D, _V = 16384, 50, 64, 1000000
_BR = 8192
_NB = 62
_S = _BR * _NB
_VP = 2 * _S

_NC, _NS, _L = 2, 16, 16
_NW = _NC * _NS
_BPW = _B // _NW
_C = 8
_CW = _C * _H
_NCH = _BPW // _C
_INV = 1.0 / _H
_WIDX = _BPW * _H


def _pack_body(x1_ref, x2_ref, out_ref):
    x = jnp.concatenate([x1_ref[...], x2_ref[...]], axis=0)
    out_ref[...] = jnp.swapaxes(x, 0, 1)


def _pack(tab_t):
    return pl.pallas_call(
        _pack_body,
        out_shape=jax.ShapeDtypeStruct((_S, 2 * _D), jnp.float32),
        grid=(_NB,),
        in_specs=[
            pl.BlockSpec((_D, _BR), lambda g: (0, g)),
            pl.BlockSpec(
                (_D, _BR),
                lambda g: (0, jnp.minimum(g + _NB, (_V + _BR - 1) // _BR - 1)),
            ),
        ],
        out_specs=pl.BlockSpec((_BR, 2 * _D), lambda g: (g, 0)),
    )(tab_t, tab_t)


def _body(xs_hbm, tab_hbm, out_hbm, xs_v, rows0, rows1, out_v, sem0, sem1):
    wid = lax.axis_index("s") * _NC + lax.axis_index("c")
    base = wid * _BPW

    pltpu.sync_copy(xs_hbm.at[pl.ds(base * _H, _WIDX)], xs_v)

    @pl.loop(0, _WIDX, step=_L)
    def _remap(k):
        v = xs_v[pl.ds(k, _L)]
        ge = v >= _S
        xs_v[pl.ds(k, _L)] = 2 * v - jnp.where(ge, _VP - 1, 0)

    pltpu.async_copy(tab_hbm.at[xs_v.at[pl.ds(0, _CW)]], rows0, sem0)
    pltpu.async_copy(tab_hbm.at[xs_v.at[pl.ds(_CW, _CW)]], rows1, sem1)

    @pl.loop(0, _NCH, step=2)
    def _chunks(ci):
        for b in range(2):
            rows = rows0 if b == 0 else rows1
            sem = sem0 if b == 0 else sem1
            cur = ci + b
            pltpu.make_async_copy(
                tab_hbm.at[xs_v.at[pl.ds(cur * _CW, _CW)]], rows, sem
            ).wait()

            @pl.loop(0, _C)
            def _items(i, rows=rows, cur=cur):
                rowbase = i * _H
                z = jnp.zeros((_L,), jnp.float32)

                @pl.loop(0, _H, init_carry=(z, z, z, z), unroll=2)
                def _acc(j, carry, rows=rows, rowbase=rowbase):
                    a0, a1, a2, a3 = carry
                    rr = rowbase + j
                    return (
                        a0 + rows[rr, pl.ds(0, _L)],
                        a1 + rows[rr, pl.ds(_L, _L)],
                        a2 + rows[rr, pl.ds(2 * _L, _L)],
                        a3 + rows[rr, pl.ds(3 * _L, _L)],
                    )

                a0, a1, a2, a3 = _acc
                r0 = cur * _C + i
                out_v[r0, pl.ds(0, _L)] = a0 * _INV
                out_v[r0, pl.ds(_L, _L)] = a1 * _INV
                out_v[r0, pl.ds(2 * _L, _L)] = a2 * _INV
                out_v[r0, pl.ds(3 * _L, _L)] = a3 * _INV

            nxt = cur + 2

            @pl.when(nxt < _NCH)
            def _fire(rows=rows, sem=sem, nxt=nxt):
                pltpu.async_copy(
                    tab_hbm.at[xs_v.at[pl.ds(nxt * _CW, _CW)]], rows, sem
                )

    pltpu.sync_copy(out_v, out_hbm.at[pl.ds(base, _BPW)])


@functools.cache
def _make_sc_kernel():
    mesh = plsc.VectorSubcoreMesh(
        core_axis_name="c", subcore_axis_name="s",
        num_cores=_NC, num_subcores=_NS,
    )
    return pl.kernel(
        _body,
        out_type=jax.ShapeDtypeStruct((_B, _D), jnp.float32),
        mesh=mesh,
        scratch_types=[
            pltpu.VMEM((_WIDX,), jnp.int32),
            pltpu.VMEM((_CW, _D), jnp.float32),
            pltpu.VMEM((_CW, _D), jnp.float32),
            pltpu.VMEM((_BPW, _D), jnp.float32),
            pltpu.SemaphoreType.DMA,
            pltpu.SemaphoreType.DMA,
        ],
        compiler_params=pltpu.CompilerParams(use_tc_tiling_on_sc=False),
    )


def kernel(xs, table):
    xs_flat = xs.reshape(-1).astype(jnp.int32)
    tab_lin = _pack(table.T).reshape(_VP, _D)
    return _make_sc_kernel()(xs_flat, tab_lin)


# speedup vs baseline: 2.6662x; 1.0189x over previous
import functools

import jax
import jax.numpy as jnp
from jax import lax
from jax.experimental import pallas as pl
from jax.experimental.pallas import tpu as pltpu
from jax.experimental.pallas import tpu_sc as plsc

_B, _H, _D, _V = 16384, 50, 64, 1000000
_BR = 16384
_NB = 31
_S = _BR * _NB
_VP = 2 * _S

_NC, _NS, _L = 2, 16, 16
_NW = _NC * _NS
_BPW = _B // _NW
_C = 8
_CW = _C * _H
_NCH = _BPW // _C
_INV = 1.0 / _H
_WIDX = _BPW * _H


def _pack_body(x1_ref, x2_ref, out_ref):
    x = jnp.concatenate([x1_ref[...], x2_ref[...]], axis=0)
    out_ref[...] = jnp.swapaxes(x, 0, 1)


def _pack(tab_t):
    return pl.pallas_call(
        _pack_body,
        out_shape=jax.ShapeDtypeStruct((_S, 2 * _D), jnp.float32),
        grid=(_NB,),
        in_specs=[
            pl.BlockSpec((_D, _BR), lambda g: (0, g)),
            pl.BlockSpec(
                (_D, _BR),
                lambda g: (0, jnp.minimum(g + _NB, (_V + _BR - 1) // _BR - 1)),
            ),
        ],
        out_specs=pl.BlockSpec((_BR, 2 * _D), lambda g: (g, 0)),
    )(tab_t, tab_t)


def _body(xs_hbm, tab_hbm, out_hbm, xs_v, rows0, rows1, out_v, sem0, sem1):
    wid = lax.axis_index("s") * _NC + lax.axis_index("c")
    base = wid * _BPW

    pltpu.sync_copy(xs_hbm.at[pl.ds(base * _H, _WIDX)], xs_v)

    @pl.loop(0, _WIDX, step=_L)
    def _remap(k):
        v = xs_v[pl.ds(k, _L)]
        ge = v >= _S
        xs_v[pl.ds(k, _L)] = 2 * v - jnp.where(ge, _VP - 1, 0)

    pltpu.async_copy(tab_hbm.at[xs_v.at[pl.ds(0, _CW)]], rows0, sem0)
    pltpu.async_copy(tab_hbm.at[xs_v.at[pl.ds(_CW, _CW)]], rows1, sem1)

    @pl.loop(0, _NCH, step=2)
    def _chunks(ci):
        for b in range(2):
            rows = rows0 if b == 0 else rows1
            sem = sem0 if b == 0 else sem1
            cur = ci + b
            pltpu.make_async_copy(
                tab_hbm.at[xs_v.at[pl.ds(cur * _CW, _CW)]], rows, sem
            ).wait()

            @pl.loop(0, _C)
            def _items(i, rows=rows, cur=cur):
                rowbase = i * _H
                z = jnp.zeros((_L,), jnp.float32)

                @pl.loop(0, _H, init_carry=(z, z, z, z), unroll=2)
                def _acc(j, carry, rows=rows, rowbase=rowbase):
                    a0, a1, a2, a3 = carry
                    rr = rowbase + j
                    return (
                        a0 + rows[rr, pl.ds(0, _L)],
                        a1 + rows[rr, pl.ds(_L, _L)],
                        a2 + rows[rr, pl.ds(2 * _L, _L)],
                        a3 + rows[rr, pl.ds(3 * _L, _L)],
                    )

                a0, a1, a2, a3 = _acc
                r0 = cur * _C + i
                out_v[r0, pl.ds(0, _L)] = a0 * _INV
                out_v[r0, pl.ds(_L, _L)] = a1 * _INV
                out_v[r0, pl.ds(2 * _L, _L)] = a2 * _INV
                out_v[r0, pl.ds(3 * _L, _L)] = a3 * _INV

            nxt = cur + 2

            @pl.when(nxt < _NCH)
            def _fire(rows=rows, sem=sem, nxt=nxt):
                pltpu.async_copy(
                    tab_hbm.at[xs_v.at[pl.ds(nxt * _CW, _CW)]], rows, sem
                )

    pltpu.sync_copy(out_v, out_hbm.at[pl.ds(base, _BPW)])


@functools.cache
def _make_sc_kernel():
    mesh = plsc.VectorSubcoreMesh(
        core_axis_name="c", subcore_axis_name="s",
        num_cores=_NC, num_subcores=_NS,
    )
    return pl.kernel(
        _body,
        out_type=jax.ShapeDtypeStruct((_B, _D), jnp.float32),
        mesh=mesh,
        scratch_types=[
            pltpu.VMEM((_WIDX,), jnp.int32),
            pltpu.VMEM((_CW, _D), jnp.float32),
            pltpu.VMEM((_CW, _D), jnp.float32),
            pltpu.VMEM((_BPW, _D), jnp.float32),
            pltpu.SemaphoreType.DMA,
            pltpu.SemaphoreType.DMA,
        ],
        compiler_params=pltpu.CompilerParams(use_tc_tiling_on_sc=False),
    )


def kernel(xs, table):
    xs_flat = xs.reshape(-1).astype(jnp.int32)
    tab_lin = _pack(table.T).reshape(_VP, _D)
    return _make_sc_kernel()(xs_flat, tab_lin)


# speedup vs baseline: 2.8421x; 1.0660x over previous
import functools

import jax
import jax.numpy as jnp
from jax import lax
from jax.experimental import pallas as pl
from jax.experimental.pallas import tpu as pltpu
from jax.experimental.pallas import tpu_sc as plsc

_B, _H, _D, _V = 16384, 50, 64, 1000000
_BR = 16384
_NB = 31
_S = _BR * _NB
_VP = 2 * _S

_NC, _NS, _L = 2, 16, 16
_NW = _NC * _NS
_BPW = _B // _NW
_C = 4
_CW = _C * _H
_NCH = _BPW // _C
_NBUF = 4
_INV = 1.0 / _H
_WIDX = _BPW * _H


def _pack_body(x1_ref, x2_ref, out_ref):
    x = jnp.concatenate([x1_ref[...], x2_ref[...]], axis=0)
    out_ref[...] = jnp.swapaxes(x, 0, 1)


def _pack(tab_t):
    return pl.pallas_call(
        _pack_body,
        out_shape=jax.ShapeDtypeStruct((_S, 2 * _D), jnp.float32),
        grid=(_NB,),
        in_specs=[
            pl.BlockSpec((_D, _BR), lambda g: (0, g)),
            pl.BlockSpec(
                (_D, _BR),
                lambda g: (0, jnp.minimum(g + _NB, (_V + _BR - 1) // _BR - 1)),
            ),
        ],
        out_specs=pl.BlockSpec((_BR, 2 * _D), lambda g: (g, 0)),
    )(tab_t, tab_t)


def _body(xs_hbm, tab_hbm, out_hbm, xs_v, rows_bufs, out_v, sems):
    wid = lax.axis_index("s") * _NC + lax.axis_index("c")
    base = wid * _BPW

    pltpu.sync_copy(xs_hbm.at[pl.ds(base * _H, _WIDX)], xs_v)

    @pl.loop(0, _WIDX, step=_L)
    def _remap(k):
        v = xs_v[pl.ds(k, _L)]
        ge = v >= _S
        xs_v[pl.ds(k, _L)] = 2 * v - jnp.where(ge, _VP - 1, 0)

    for b in range(_NBUF):
        pltpu.async_copy(
            tab_hbm.at[xs_v.at[pl.ds(b * _CW, _CW)]], rows_bufs[b], sems[b]
        )

    @pl.loop(0, _NCH, step=_NBUF)
    def _chunks(ci):
        for b in range(_NBUF):
            rows = rows_bufs[b]
            sem = sems[b]
            cur = ci + b
            pltpu.make_async_copy(
                tab_hbm.at[xs_v.at[pl.ds(cur * _CW, _CW)]], rows, sem
            ).wait()

            @pl.loop(0, _C)
            def _items(i, rows=rows, cur=cur):
                rowbase = i * _H
                z = jnp.zeros((_L,), jnp.float32)

                @pl.loop(0, _H, init_carry=(z, z, z, z), unroll=5)
                def _acc(j, carry, rows=rows, rowbase=rowbase):
                    a0, a1, a2, a3 = carry
                    rr = rowbase + j
                    return (
                        a0 + rows[rr, pl.ds(0, _L)],
                        a1 + rows[rr, pl.ds(_L, _L)],
                        a2 + rows[rr, pl.ds(2 * _L, _L)],
                        a3 + rows[rr, pl.ds(3 * _L, _L)],
                    )

                a0, a1, a2, a3 = _acc
                r0 = cur * _C + i
                out_v[r0, pl.ds(0, _L)] = a0 * _INV
                out_v[r0, pl.ds(_L, _L)] = a1 * _INV
                out_v[r0, pl.ds(2 * _L, _L)] = a2 * _INV
                out_v[r0, pl.ds(3 * _L, _L)] = a3 * _INV

            nxt = cur + _NBUF

            @pl.when(nxt < _NCH)
            def _fire(rows=rows, sem=sem, nxt=nxt):
                pltpu.async_copy(
                    tab_hbm.at[xs_v.at[pl.ds(nxt * _CW, _CW)]], rows, sem
                )

    pltpu.sync_copy(out_v, out_hbm.at[pl.ds(base, _BPW)])


@functools.cache
def _make_sc_kernel():
    mesh = plsc.VectorSubcoreMesh(
        core_axis_name="c", subcore_axis_name="s",
        num_cores=_NC, num_subcores=_NS,
    )

    def body(xs_hbm, tab_hbm, out_hbm, xs_v,
             r0, r1, r2, r3, out_v, s0, s1, s2, s3):
        _body(xs_hbm, tab_hbm, out_hbm, xs_v, (r0, r1, r2, r3), out_v,
              (s0, s1, s2, s3))

    return pl.kernel(
        body,
        out_type=jax.ShapeDtypeStruct((_B, _D), jnp.float32),
        mesh=mesh,
        scratch_types=[
            pltpu.VMEM((_WIDX,), jnp.int32),
            pltpu.VMEM((_CW, _D), jnp.float32),
            pltpu.VMEM((_CW, _D), jnp.float32),
            pltpu.VMEM((_CW, _D), jnp.float32),
            pltpu.VMEM((_CW, _D), jnp.float32),
            pltpu.VMEM((_BPW, _D), jnp.float32),
            pltpu.SemaphoreType.DMA,
            pltpu.SemaphoreType.DMA,
            pltpu.SemaphoreType.DMA,
            pltpu.SemaphoreType.DMA,
        ],
        compiler_params=pltpu.CompilerParams(use_tc_tiling_on_sc=False),
    )


def kernel(xs, table):
    xs_flat = xs.reshape(-1).astype(jnp.int32)
    tab_lin = _pack(table.T).reshape(_VP, _D)
    return _make_sc_kernel()(xs_flat, tab_lin)
